# baseline (speedup 1.0000x reference)
import functools

import jax
import jax.numpy as jnp
from jax import lax
from jax.experimental import pallas as pl
from jax.experimental.pallas import tpu as pltpu
from jax.experimental.pallas import tpu_sc as plsc

N = 10000
E = 320000
D = 128
NC = 2
NS = 16
NW = NC * NS
EPW = E // NW
CH = 40
NCHUNK = EPW // CH
NBUF = 5
NROUND = NCHUNK // NBUF
N2 = 10240
RPT = N2 // NS


def _sc_agg_body(m_hbm, src_hbm, dst_hbm, s_out,
                 src_v, dst_v, rows_v, acc_sh, semg, sems):
    c = lax.axis_index("c")
    s = lax.axis_index("s")
    wid = c * NS + s

    def _zrow(i, _):
        for j in range(8):
            rows_v[0, i, pl.ds(j * 16, 16)] = jnp.zeros((16,), jnp.float32)
        return 0
    lax.fori_loop(0, CH, _zrow, 0)
    for k in range(RPT // CH):
        pltpu.sync_copy(rows_v.at[0], acc_sh.at[pl.ds(s * RPT + k * CH, CH)])

    pltpu.sync_copy(src_hbm.at[wid], src_v)
    pltpu.sync_copy(dst_hbm.at[wid], dst_v)

    plsc.subcore_barrier()

    def _g_start(ci, b):
        pltpu.async_copy(m_hbm.at[src_v.at[ci]], rows_v.at[b], semg[b])

    def _g_wait(b):
        pltpu.make_async_copy(m_hbm.at[pl.ds(0, CH)], rows_v.at[b],
                              semg[b]).wait()

    def _s_start(ci, b):
        pltpu.async_copy(rows_v.at[b], acc_sh.at[dst_v.at[ci]], sems[b],
                         add=True)

    def _s_wait(b):
        pltpu.make_async_copy(rows_v.at[b], acc_sh.at[pl.ds(0, CH)],
                              sems[b]).wait()

    for b in range(NBUF):
        _g_start(b, b)

    def _round(k, _):
        c0 = k * NBUF
        for b in range(NBUF):
            _g_wait(b)
            _s_start(c0 + b, b)
        for b in range(NBUF):
            _s_wait(b)
            _g_start(c0 + NBUF + b, b)
        return 0
    lax.fori_loop(0, NROUND - 1, _round, 0)

    c0 = (NROUND - 1) * NBUF
    for b in range(NBUF):
        _g_wait(b)
        _s_start(c0 + b, b)
    for b in range(NBUF):
        _s_wait(b)

    plsc.subcore_barrier()

    pltpu.sync_copy(acc_sh.at[pl.ds(s * RPT, RPT)],
                    s_out.at[c, pl.ds(s * RPT, RPT)])


def _sc_deg_body(dst_hbm, deg_out, dst_v, ones_v, zbufd_v, dega_sh):
    c = lax.axis_index("c")
    s = lax.axis_index("s")
    wid = c * NS + s

    def _zdrow(i, _):
        zbufd_v[i, pl.ds(0, 16)] = jnp.zeros((16,), jnp.float32)
        return 0
    lax.fori_loop(0, RPT, _zdrow, 0)
    pltpu.sync_copy(zbufd_v, dega_sh.at[pl.ds(s * RPT, RPT)])

    def _orow(i, _):
        ones_v[i, pl.ds(0, 16)] = jnp.ones((16,), jnp.float32)
        return 0
    lax.fori_loop(0, CH, _orow, 0)

    pltpu.sync_copy(dst_hbm.at[wid], dst_v)

    plsc.subcore_barrier()

    def _chunk(ci, _):
        pltpu.sync_copy(ones_v, dega_sh.at[dst_v.at[ci]], add=True)
        return 0
    lax.fori_loop(0, NCHUNK, _chunk, 0)

    plsc.subcore_barrier()

    pltpu.sync_copy(dega_sh.at[pl.ds(s * RPT, RPT)],
                    deg_out.at[c, pl.ds(s * RPT, RPT)])


_SC_MESH = plsc.VectorSubcoreMesh(core_axis_name="c", subcore_axis_name="s")
_SC_PARAMS = pltpu.CompilerParams(use_tc_tiling_on_sc=False)

_sc_deg = pl.kernel(
    _sc_deg_body,
    out_type=jax.ShapeDtypeStruct((NC, N2, 16), jnp.float32),
    mesh=_SC_MESH,
    compiler_params=_SC_PARAMS,
    scratch_types=[
        pltpu.VMEM((NCHUNK, CH), jnp.int32),
        pltpu.VMEM((CH, 16), jnp.float32),
        pltpu.VMEM((RPT, 16), jnp.float32),
        pltpu.VMEM_SHARED((N2, 16), jnp.float32),
    ],
)

_sc_agg = pl.kernel(
    _sc_agg_body,
    out_type=jax.ShapeDtypeStruct((NC, N2, 128), jnp.float32),
    mesh=_SC_MESH,
    compiler_params=_SC_PARAMS,
    scratch_types=[
        pltpu.VMEM((NCHUNK, CH), jnp.int32),
        pltpu.VMEM((NCHUNK, CH), jnp.int32),
        pltpu.VMEM((NBUF, CH, 128), jnp.float32),
        pltpu.VMEM_SHARED((N2, 128), jnp.float32),
        [pltpu.SemaphoreType.DMA] * NBUF,
        [pltpu.SemaphoreType.DMA] * NBUF,
    ],
)


def _tc0_body(x_ref, wl_ref, wr_ref, bl_ref, m_ref, r_ref):
    x = x_ref[...]
    m_ref[...] = jnp.dot(x, wl_ref[...], preferred_element_type=jnp.float32)
    r_ref[...] = (jnp.dot(x, wr_ref[...], preferred_element_type=jnp.float32)
                  + bl_ref[...])


def _bn_relu(s_ref, dg_ref, r_ref, g_ref, b_ref):
    sp = s_ref[...]
    s = sp[0, :N] + sp[1, :N]
    dg = dg_ref[...]
    deg = dg[0, :N, 0:1] + dg[1, :N, 0:1]
    a = s / jnp.maximum(deg, 1.0) + r_ref[...]
    mean = jnp.mean(a, axis=0, keepdims=True)
    var = jnp.mean((a - mean) ** 2, axis=0, keepdims=True)
    h = (a - mean) * lax.rsqrt(var + 1e-5) * g_ref[...] + b_ref[...]
    return jnp.maximum(h, 0.0)


def _tc_mid_body(s_ref, dg_ref, r_ref, g_ref, b_ref, wl_ref, bln_ref, wr_ref,
                 m_ref, rn_ref):
    h = _bn_relu(s_ref, dg_ref, r_ref, g_ref, b_ref)
    m_ref[...] = jnp.dot(h, wl_ref[...], preferred_element_type=jnp.float32)
    rn_ref[...] = (jnp.dot(h, wr_ref[...], preferred_element_type=jnp.float32)
                   + bln_ref[...])


def _tc_fin_body(s_ref, dg_ref, r_ref, g_ref, b_ref, wc1_ref, bc1_ref,
                 wc2_ref, bc2_ref, o_ref):
    h = _bn_relu(s_ref, dg_ref, r_ref, g_ref, b_ref)
    o1 = jnp.maximum(
        jnp.dot(h, wc1_ref[...], preferred_element_type=jnp.float32)
        + bc1_ref[...], 0.0)
    o_ref[...] = (jnp.dot(o1, wc2_ref[...], preferred_element_type=jnp.float32)
                  + bc2_ref[...])


_f32 = jnp.float32


def _tc0(x, wl, wr, bl):
    return pl.pallas_call(
        _tc0_body,
        out_shape=[jax.ShapeDtypeStruct((N, 128), _f32)] * 2,
    )(x, wl, wr, bl)


def _tc_mid(s_par, deg_par, r, g, b, wl, bln, wr):
    return pl.pallas_call(
        _tc_mid_body,
        out_shape=[jax.ShapeDtypeStruct((N, 128), _f32)] * 2,
    )(s_par, deg_par, r, g, b, wl, bln, wr)


def _tc_fin(s_par, deg_par, r, g, b, wc1, bc1, wc2, bc2):
    return pl.pallas_call(
        _tc_fin_body,
        out_shape=jax.ShapeDtypeStruct((N, 1), _f32),
    )(s_par, deg_par, r, g, b, wc1, bc1, wc2, bc2)


def kernel(x, edge_index, Wl0, bl0, Wr0, gamma0, beta0, Wl1, bl1, Wr1,
           gamma1, beta1, Wl2, bl2, Wr2, gamma2, beta2, Wc1, bc1, Wc2, bc2):
    src = edge_index[0].reshape(NW, NCHUNK, CH)
    dst = edge_index[1].reshape(NW, NCHUNK, CH)
    row = lambda v: v.reshape(1, -1)

    m, r = _tc0(x, Wl0, Wr0, row(bl0))
    deg_par = _sc_deg(dst)
    s_par = _sc_agg(m, src, dst)
    m, r = _tc_mid(s_par, deg_par, r, row(gamma0), row(beta0),
                   Wl1, row(bl1), Wr1)
    s_par = _sc_agg(m, src, dst)
    m, r = _tc_mid(s_par, deg_par, r, row(gamma1), row(beta1),
                   Wl2, row(bl2), Wr2)
    s_par = _sc_agg(m, src, dst)
    out = _tc_fin(s_par, deg_par, r, row(gamma2), row(beta2),
                  Wc1, row(bc1), Wc2, bc2.reshape(1, 1))
    return out[:, 0]


# speedup vs baseline: 1.0095x; 1.0095x over previous
import functools

import jax
import jax.numpy as jnp
from jax import lax
from jax.experimental import pallas as pl
from jax.experimental.pallas import tpu as pltpu
from jax.experimental.pallas import tpu_sc as plsc

N = 10000
E = 320000
D = 128
NC = 2
NS = 16
NW = NC * NS
EPW = E // NW
CH = 40
NCHUNK = EPW // CH
NBUF = 5
NROUND = NCHUNK // NBUF
N2 = 10240
RPT = N2 // NS


def _sc_agg_body(m_hbm, src_hbm, dst_hbm, s_out,
                 src_v, dst_v, rows_v, acc_sh, semg, sems):
    c = lax.axis_index("c")
    s = lax.axis_index("s")
    wid = c * NS + s

    pltpu.sync_copy(src_hbm.at[wid], src_v)
    pltpu.sync_copy(dst_hbm.at[wid], dst_v)

    def _g_start(ci, b):
        pltpu.async_copy(m_hbm.at[src_v.at[ci]], rows_v.at[b], semg[b])

    def _g_wait(b):
        pltpu.make_async_copy(m_hbm.at[pl.ds(0, CH)], rows_v.at[b],
                              semg[b]).wait()

    def _s_start(ci, b):
        pltpu.async_copy(rows_v.at[b], acc_sh.at[dst_v.at[ci]], sems[b],
                         add=True)

    def _s_wait(b):
        pltpu.make_async_copy(rows_v.at[b], acc_sh.at[pl.ds(0, CH)],
                              sems[b]).wait()

    for b in range(1, NBUF):
        _g_start(b, b)

    def _zrow(i, _):
        for j in range(8):
            rows_v[0, i, pl.ds(j * 16, 16)] = jnp.zeros((16,), jnp.float32)
        return 0
    lax.fori_loop(0, CH, _zrow, 0)
    for k in range(RPT // CH):
        pltpu.sync_copy(rows_v.at[0], acc_sh.at[pl.ds(s * RPT + k * CH, CH)])

    plsc.subcore_barrier()

    _g_start(0, 0)

    def _round(k, _):
        c0 = k * NBUF
        for b in range(NBUF):
            _g_wait(b)
            _s_start(c0 + b, b)
        for b in range(NBUF):
            _s_wait(b)
            _g_start(c0 + NBUF + b, b)
        return 0
    lax.fori_loop(0, NROUND - 1, _round, 0)

    c0 = (NROUND - 1) * NBUF
    for b in range(NBUF):
        _g_wait(b)
        _s_start(c0 + b, b)
    for b in range(NBUF):
        _s_wait(b)

    plsc.subcore_barrier()

    pltpu.sync_copy(acc_sh.at[pl.ds(s * RPT, RPT)],
                    s_out.at[c, pl.ds(s * RPT, RPT)])


def _sc_deg_body(dst_hbm, deg_out, dst_v, ones_v, zbufd_v, dega_sh):
    c = lax.axis_index("c")
    s = lax.axis_index("s")
    wid = c * NS + s

    def _zdrow(i, _):
        zbufd_v[i, pl.ds(0, 16)] = jnp.zeros((16,), jnp.float32)
        return 0
    lax.fori_loop(0, RPT, _zdrow, 0)
    pltpu.sync_copy(zbufd_v, dega_sh.at[pl.ds(s * RPT, RPT)])

    def _orow(i, _):
        ones_v[i, pl.ds(0, 16)] = jnp.ones((16,), jnp.float32)
        return 0
    lax.fori_loop(0, CH, _orow, 0)

    pltpu.sync_copy(dst_hbm.at[wid], dst_v)

    plsc.subcore_barrier()

    def _chunk(ci, _):
        pltpu.sync_copy(ones_v, dega_sh.at[dst_v.at[ci]], add=True)
        return 0
    lax.fori_loop(0, NCHUNK, _chunk, 0)

    plsc.subcore_barrier()

    pltpu.sync_copy(dega_sh.at[pl.ds(s * RPT, RPT)],
                    deg_out.at[c, pl.ds(s * RPT, RPT)])


_SC_MESH = plsc.VectorSubcoreMesh(core_axis_name="c", subcore_axis_name="s")
_SC_PARAMS = pltpu.CompilerParams(use_tc_tiling_on_sc=False)

_sc_deg = pl.kernel(
    _sc_deg_body,
    out_type=jax.ShapeDtypeStruct((NC, N2, 16), jnp.float32),
    mesh=_SC_MESH,
    compiler_params=_SC_PARAMS,
    scratch_types=[
        pltpu.VMEM((NCHUNK, CH), jnp.int32),
        pltpu.VMEM((CH, 16), jnp.float32),
        pltpu.VMEM((RPT, 16), jnp.float32),
        pltpu.VMEM_SHARED((N2, 16), jnp.float32),
    ],
)

_sc_agg = pl.kernel(
    _sc_agg_body,
    out_type=jax.ShapeDtypeStruct((NC, N2, 128), jnp.float32),
    mesh=_SC_MESH,
    compiler_params=_SC_PARAMS,
    scratch_types=[
        pltpu.VMEM((NCHUNK, CH), jnp.int32),
        pltpu.VMEM((NCHUNK, CH), jnp.int32),
        pltpu.VMEM((NBUF, CH, 128), jnp.float32),
        pltpu.VMEM_SHARED((N2, 128), jnp.float32),
        [pltpu.SemaphoreType.DMA] * NBUF,
        [pltpu.SemaphoreType.DMA] * NBUF,
    ],
)


def _tc0_body(x_ref, wl_ref, wr_ref, bl_ref, m_ref, r_ref):
    x = x_ref[...]
    m_ref[...] = jnp.dot(x, wl_ref[...], preferred_element_type=jnp.float32)
    r_ref[...] = (jnp.dot(x, wr_ref[...], preferred_element_type=jnp.float32)
                  + bl_ref[...])


def _bn_relu(s_ref, dinv, r_ref, g_ref, b_ref):
    sp = s_ref[...]
    s = sp[0, :N] + sp[1, :N]
    a = s * dinv + r_ref[...]
    mean = jnp.mean(a, axis=0, keepdims=True)
    var = jnp.mean((a - mean) ** 2, axis=0, keepdims=True)
    h = (a - mean) * lax.rsqrt(var + 1e-5) * g_ref[...] + b_ref[...]
    return jnp.maximum(h, 0.0)


def _tc_mid0_body(s_ref, dg_ref, r_ref, g_ref, b_ref, wl_ref, bln_ref,
                  wr_ref, m_ref, rn_ref, dinv_ref):
    dg = dg_ref[...]
    deg = dg[0, :N, 0:1] + dg[1, :N, 0:1]
    dinv = 1.0 / jnp.maximum(deg, 1.0)
    dinv_ref[...] = dinv
    h = _bn_relu(s_ref, dinv, r_ref, g_ref, b_ref)
    m_ref[...] = jnp.dot(h, wl_ref[...], preferred_element_type=jnp.float32)
    rn_ref[...] = (jnp.dot(h, wr_ref[...], preferred_element_type=jnp.float32)
                   + bln_ref[...])


def _tc_mid_body(s_ref, dinv_ref, r_ref, g_ref, b_ref, wl_ref, bln_ref,
                 wr_ref, m_ref, rn_ref):
    h = _bn_relu(s_ref, dinv_ref[...], r_ref, g_ref, b_ref)
    m_ref[...] = jnp.dot(h, wl_ref[...], preferred_element_type=jnp.float32)
    rn_ref[...] = (jnp.dot(h, wr_ref[...], preferred_element_type=jnp.float32)
                   + bln_ref[...])


def _tc_fin_body(s_ref, dinv_ref, r_ref, g_ref, b_ref, wc1_ref, bc1_ref,
                 wc2_ref, bc2_ref, o_ref):
    h = _bn_relu(s_ref, dinv_ref[...], r_ref, g_ref, b_ref)
    o1 = jnp.maximum(
        jnp.dot(h, wc1_ref[...], preferred_element_type=jnp.float32)
        + bc1_ref[...], 0.0)
    o_ref[...] = (jnp.dot(o1, wc2_ref[...], preferred_element_type=jnp.float32)
                  + bc2_ref[...])


_f32 = jnp.float32


def _tc0(x, wl, wr, bl):
    return pl.pallas_call(
        _tc0_body,
        out_shape=[jax.ShapeDtypeStruct((N, 128), _f32)] * 2,
    )(x, wl, wr, bl)


def _tc_mid0(s_par, deg_par, r, g, b, wl, bln, wr):
    return pl.pallas_call(
        _tc_mid0_body,
        out_shape=[jax.ShapeDtypeStruct((N, 128), _f32),
                   jax.ShapeDtypeStruct((N, 128), _f32),
                   jax.ShapeDtypeStruct((N, 1), _f32)],
    )(s_par, deg_par, r, g, b, wl, bln, wr)


def _tc_mid(s_par, dinv, r, g, b, wl, bln, wr):
    return pl.pallas_call(
        _tc_mid_body,
        out_shape=[jax.ShapeDtypeStruct((N, 128), _f32)] * 2,
    )(s_par, dinv, r, g, b, wl, bln, wr)


def _tc_fin(s_par, dinv, r, g, b, wc1, bc1, wc2, bc2):
    return pl.pallas_call(
        _tc_fin_body,
        out_shape=jax.ShapeDtypeStruct((N, 1), _f32),
    )(s_par, dinv, r, g, b, wc1, bc1, wc2, bc2)


def kernel(x, edge_index, Wl0, bl0, Wr0, gamma0, beta0, Wl1, bl1, Wr1,
           gamma1, beta1, Wl2, bl2, Wr2, gamma2, beta2, Wc1, bc1, Wc2, bc2):
    src = edge_index[0].reshape(NW, NCHUNK, CH)
    dst = edge_index[1].reshape(NW, NCHUNK, CH)
    row = lambda v: v.reshape(1, -1)

    m, r = _tc0(x, Wl0, Wr0, row(bl0))
    deg_par = _sc_deg(dst)
    s_par = _sc_agg(m, src, dst)
    m, r, dinv = _tc_mid0(s_par, deg_par, r, row(gamma0), row(beta0),
                          Wl1, row(bl1), Wr1)
    s_par = _sc_agg(m, src, dst)
    m, r = _tc_mid(s_par, dinv, r, row(gamma1), row(beta1),
                   Wl2, row(bl2), Wr2)
    s_par = _sc_agg(m, src, dst)
    out = _tc_fin(s_par, dinv, r, row(gamma2), row(beta2),
                  Wc1, row(bc1), Wc2, bc2.reshape(1, 1))
    return out[:, 0]


# speedup vs baseline: 1.0140x; 1.0044x over previous
import functools

import jax
import jax.numpy as jnp
from jax import lax
from jax.experimental import pallas as pl
from jax.experimental.pallas import tpu as pltpu
from jax.experimental.pallas import tpu_sc as plsc

N = 10000
E = 320000
D = 128
NC = 2
NS = 16
NW = NC * NS
EPW = E // NW
CH = 40
NCHUNK = EPW // CH
NBUF = 5
NROUND = NCHUNK // NBUF
N2 = 10240
RPT = N2 // NS


def _sc_agg_body(m_hbm, src_hbm, dst_hbm, s_out,
                 src_v, dst_v, rows_v, acc_sh, semg, sems):
    c = lax.axis_index("c")
    s = lax.axis_index("s")
    wid = c * NS + s

    pltpu.sync_copy(src_hbm.at[wid], src_v)
    pltpu.sync_copy(dst_hbm.at[wid], dst_v)

    def _g_start(ci, b):
        pltpu.async_copy(m_hbm.at[src_v.at[ci]], rows_v.at[b], semg[b])

    def _g_wait(b):
        pltpu.make_async_copy(m_hbm.at[pl.ds(0, CH)], rows_v.at[b],
                              semg[b]).wait()

    def _s_start(ci, b):
        pltpu.async_copy(rows_v.at[b], acc_sh.at[dst_v.at[ci]], sems[b],
                         add=True)

    def _s_wait(b):
        pltpu.make_async_copy(rows_v.at[b], acc_sh.at[pl.ds(0, CH)],
                              sems[b]).wait()

    for b in range(1, NBUF):
        _g_start(b, b)

    def _zrow(i, _):
        for j in range(8):
            rows_v[0, i, pl.ds(j * 16, 16)] = jnp.zeros((16,), jnp.float32)
        return 0
    lax.fori_loop(0, CH, _zrow, 0)
    for k in range(RPT // CH):
        pltpu.sync_copy(rows_v.at[0], acc_sh.at[pl.ds(s * RPT + k * CH, CH)])

    plsc.subcore_barrier()

    _g_start(0, 0)

    def _round(k, _):
        c0 = k * NBUF
        for b in range(NBUF):
            _g_wait(b)
            _s_start(c0 + b, b)
        for b in range(NBUF):
            _s_wait(b)
            _g_start(c0 + NBUF + b, b)
        return 0
    lax.fori_loop(0, NROUND - 1, _round, 0)

    c0 = (NROUND - 1) * NBUF
    for b in range(NBUF):
        _g_wait(b)
        _s_start(c0 + b, b)
    for b in range(NBUF):
        _s_wait(b)

    plsc.subcore_barrier()

    pltpu.sync_copy(acc_sh.at[pl.ds(s * RPT, RPT)],
                    s_out.at[c, pl.ds(s * RPT, RPT)])


def _sc_deg_body(dst_hbm, deg_out, dst_v, ones_v, zbufd_v, dega_sh):
    c = lax.axis_index("c")
    s = lax.axis_index("s")
    wid = c * NS + s

    def _zdrow(i, _):
        zbufd_v[i, pl.ds(0, 16)] = jnp.zeros((16,), jnp.float32)
        return 0
    lax.fori_loop(0, RPT, _zdrow, 0)
    pltpu.sync_copy(zbufd_v, dega_sh.at[pl.ds(s * RPT, RPT)])

    def _orow(i, _):
        ones_v[i, pl.ds(0, 16)] = jnp.ones((16,), jnp.float32)
        return 0
    lax.fori_loop(0, CH, _orow, 0)

    pltpu.sync_copy(dst_hbm.at[wid], dst_v)

    plsc.subcore_barrier()

    def _chunk(ci, _):
        pltpu.sync_copy(ones_v, dega_sh.at[dst_v.at[ci]], add=True)
        return 0
    lax.fori_loop(0, NCHUNK, _chunk, 0)

    plsc.subcore_barrier()

    pltpu.sync_copy(dega_sh.at[pl.ds(s * RPT, RPT)],
                    deg_out.at[c, pl.ds(s * RPT, RPT)])


_SC_MESH = plsc.VectorSubcoreMesh(core_axis_name="c", subcore_axis_name="s")
_SC_PARAMS = pltpu.CompilerParams(use_tc_tiling_on_sc=False)

_sc_deg = pl.kernel(
    _sc_deg_body,
    out_type=jax.ShapeDtypeStruct((NC, N2, 16), jnp.float32),
    mesh=_SC_MESH,
    compiler_params=_SC_PARAMS,
    scratch_types=[
        pltpu.VMEM((NCHUNK, CH), jnp.int32),
        pltpu.VMEM((CH, 16), jnp.float32),
        pltpu.VMEM((RPT, 16), jnp.float32),
        pltpu.VMEM_SHARED((N2, 16), jnp.float32),
    ],
)

_sc_agg = pl.kernel(
    _sc_agg_body,
    out_type=jax.ShapeDtypeStruct((NC, N2, 128), jnp.float32),
    mesh=_SC_MESH,
    compiler_params=_SC_PARAMS,
    scratch_types=[
        pltpu.VMEM((NCHUNK, CH), jnp.int32),
        pltpu.VMEM((NCHUNK, CH), jnp.int32),
        pltpu.VMEM((NBUF, CH, 128), jnp.float32),
        pltpu.VMEM_SHARED((N2, 128), jnp.float32),
        [pltpu.SemaphoreType.DMA] * NBUF,
        [pltpu.SemaphoreType.DMA] * NBUF,
    ],
)


def _sage_layer(s_ref, dinv, h_ref, wl_ref, bl_ref, wr_ref, g_ref, b_ref):
    sp = s_ref[...]
    agg = (sp[0, :N] + sp[1, :N]) * dinv
    h = h_ref[...]
    a = (jnp.dot(agg, wl_ref[...], preferred_element_type=jnp.float32)
         + bl_ref[...]
         + jnp.dot(h, wr_ref[...], preferred_element_type=jnp.float32))
    mean = jnp.mean(a, axis=0, keepdims=True)
    var = jnp.mean((a - mean) ** 2, axis=0, keepdims=True)
    hn = (a - mean) * lax.rsqrt(var + 1e-5) * g_ref[...] + b_ref[...]
    return jnp.maximum(hn, 0.0)


def _tc_l0_body(s_ref, dg_ref, h_ref, wl_ref, bl_ref, wr_ref, g_ref, b_ref,
                hn_ref, dinv_ref):
    dg = dg_ref[...]
    deg = dg[0, :N, 0:1] + dg[1, :N, 0:1]
    dinv = 1.0 / jnp.maximum(deg, 1.0)
    dinv_ref[...] = dinv
    hn_ref[...] = _sage_layer(s_ref, dinv, h_ref, wl_ref, bl_ref, wr_ref,
                              g_ref, b_ref)


def _tc_mid_body(s_ref, dinv_ref, h_ref, wl_ref, bl_ref, wr_ref, g_ref,
                 b_ref, hn_ref):
    hn_ref[...] = _sage_layer(s_ref, dinv_ref[...], h_ref, wl_ref, bl_ref,
                              wr_ref, g_ref, b_ref)


def _tc_fin_body(s_ref, dinv_ref, h_ref, wl_ref, bl_ref, wr_ref, g_ref,
                 b_ref, wc1_ref, bc1_ref, wc2_ref, bc2_ref, o_ref):
    h = _sage_layer(s_ref, dinv_ref[...], h_ref, wl_ref, bl_ref, wr_ref,
                    g_ref, b_ref)
    o1 = jnp.maximum(
        jnp.dot(h, wc1_ref[...], preferred_element_type=jnp.float32)
        + bc1_ref[...], 0.0)
    o_ref[...] = (jnp.dot(o1, wc2_ref[...], preferred_element_type=jnp.float32)
                  + bc2_ref[...])


_f32 = jnp.float32


def _tc_l0(s_par, deg_par, h, wl, bl, wr, g, b):
    return pl.pallas_call(
        _tc_l0_body,
        out_shape=[jax.ShapeDtypeStruct((N, 128), _f32),
                   jax.ShapeDtypeStruct((N, 1), _f32)],
    )(s_par, deg_par, h, wl, bl, wr, g, b)


def _tc_mid(s_par, dinv, h, wl, bl, wr, g, b):
    return pl.pallas_call(
        _tc_mid_body,
        out_shape=jax.ShapeDtypeStruct((N, 128), _f32),
    )(s_par, dinv, h, wl, bl, wr, g, b)


def _tc_fin(s_par, dinv, h, wl, bl, wr, g, b, wc1, bc1, wc2, bc2):
    return pl.pallas_call(
        _tc_fin_body,
        out_shape=jax.ShapeDtypeStruct((N, 1), _f32),
    )(s_par, dinv, h, wl, bl, wr, g, b, wc1, bc1, wc2, bc2)


def kernel(x, edge_index, Wl0, bl0, Wr0, gamma0, beta0, Wl1, bl1, Wr1,
           gamma1, beta1, Wl2, bl2, Wr2, gamma2, beta2, Wc1, bc1, Wc2, bc2):
    src = edge_index[0].reshape(NW, NCHUNK, CH)
    dst = edge_index[1].reshape(NW, NCHUNK, CH)
    row = lambda v: v.reshape(1, -1)

    deg_par = _sc_deg(dst)
    s_par = _sc_agg(x, src, dst)
    h, dinv = _tc_l0(s_par, deg_par, x, Wl0, row(bl0), Wr0,
                     row(gamma0), row(beta0))
    s_par = _sc_agg(h, src, dst)
    h = _tc_mid(s_par, dinv, h, Wl1, row(bl1), Wr1, row(gamma1), row(beta1))
    s_par = _sc_agg(h, src, dst)
    out = _tc_fin(s_par, dinv, h, Wl2, row(bl2), Wr2, row(gamma2),
                  row(beta2), Wc1, row(bc1), Wc2, bc2.reshape(1, 1))
    return out[:, 0]


# speedup vs baseline: 1.0349x; 1.0206x over previous
import functools

import jax
import jax.numpy as jnp
from jax import lax
from jax.experimental import pallas as pl
from jax.experimental.pallas import tpu as pltpu
from jax.experimental.pallas import tpu_sc as plsc

N = 10000
E = 320000
D = 128
NC = 2
NS = 16
NW = NC * NS
EPW = E // NW
CH = 40
NCHUNK = EPW // CH
NBUF = 5
NROUND = NCHUNK // NBUF
N2 = 10240
RPT = N2 // NS


def _sc_agg_body(m_hbm, src_hbm, dst_hbm, s_out,
                 src_v, dst_v, rows_v, acc_sh, semg, sems):
    c = lax.axis_index("c")
    s = lax.axis_index("s")
    wid = c * NS + s

    pltpu.sync_copy(src_hbm.at[wid], src_v)
    pltpu.sync_copy(dst_hbm.at[wid], dst_v)

    def _g_start(ci, b):
        pltpu.async_copy(m_hbm.at[src_v.at[ci]], rows_v.at[b], semg[b])

    def _g_wait(b):
        pltpu.make_async_copy(m_hbm.at[pl.ds(0, CH)], rows_v.at[b],
                              semg[b]).wait()

    def _s_start(ci, b):
        pltpu.async_copy(rows_v.at[b], acc_sh.at[dst_v.at[ci]], sems[b],
                         add=True)

    def _s_wait(b):
        pltpu.make_async_copy(rows_v.at[b], acc_sh.at[pl.ds(0, CH)],
                              sems[b]).wait()

    for b in range(1, NBUF):
        _g_start(b, b)

    def _zrow(i, _):
        for j in range(8):
            rows_v[0, i, pl.ds(j * 16, 16)] = jnp.zeros((16,), jnp.float32)
        return 0
    lax.fori_loop(0, CH, _zrow, 0)
    for k in range(RPT // CH):
        pltpu.sync_copy(rows_v.at[0], acc_sh.at[pl.ds(s * RPT + k * CH, CH)])

    plsc.subcore_barrier()

    _g_start(0, 0)

    def _round(k, _):
        c0 = k * NBUF
        for b in range(NBUF):
            _g_wait(b)
            _s_start(c0 + b, b)
        for b in range(NBUF):
            _s_wait(b)
            _g_start(c0 + NBUF + b, b)
        return 0
    lax.fori_loop(0, NROUND - 1, _round, 0)

    c0 = (NROUND - 1) * NBUF
    for b in range(NBUF):
        _g_wait(b)
        _s_start(c0 + b, b)
    for b in range(NBUF):
        _s_wait(b)

    plsc.subcore_barrier()

    pltpu.sync_copy(acc_sh.at[pl.ds(s * RPT, RPT)],
                    s_out.at[c, pl.ds(s * RPT, RPT)])


def _sc_deg_body(dst_hbm, deg_out, dst_v, ones_v, zbufd_v, dega_sh,
                 semd0, semd1):
    c = lax.axis_index("c")
    s = lax.axis_index("s")
    wid = c * NS + s

    def _zdrow(i, _):
        zbufd_v[i, pl.ds(0, 16)] = jnp.zeros((16,), jnp.float32)
        return 0
    lax.fori_loop(0, RPT, _zdrow, 0)
    pltpu.sync_copy(zbufd_v, dega_sh.at[pl.ds(s * RPT, RPT)])

    def _orow(i, _):
        ones_v[i, pl.ds(0, 16)] = jnp.ones((16,), jnp.float32)
        return 0
    lax.fori_loop(0, CH, _orow, 0)

    pltpu.sync_copy(dst_hbm.at[wid], dst_v)

    plsc.subcore_barrier()

    def _d_start(ci, sem):
        pltpu.async_copy(ones_v, dega_sh.at[dst_v.at[ci]], sem, add=True)

    def _d_wait(sem):
        pltpu.make_async_copy(ones_v, dega_sh.at[pl.ds(0, CH)], sem).wait()

    _d_start(0, semd0)

    def _chunk(k, _):
        c0 = k * 2
        _d_start(c0 + 1, semd1)
        _d_wait(semd0)
        _d_start(c0 + 2, semd0)
        _d_wait(semd1)
        return 0
    lax.fori_loop(0, (NCHUNK - 2) // 2, _chunk, 0)

    _d_start(NCHUNK - 1, semd1)
    _d_wait(semd0)
    _d_wait(semd1)

    plsc.subcore_barrier()

    pltpu.sync_copy(dega_sh.at[pl.ds(s * RPT, RPT)],
                    deg_out.at[c, pl.ds(s * RPT, RPT)])


_SC_MESH = plsc.VectorSubcoreMesh(core_axis_name="c", subcore_axis_name="s")
_SC_PARAMS = pltpu.CompilerParams(use_tc_tiling_on_sc=False)

_sc_deg = pl.kernel(
    _sc_deg_body,
    out_type=jax.ShapeDtypeStruct((NC, N2, 16), jnp.float32),
    mesh=_SC_MESH,
    compiler_params=_SC_PARAMS,
    scratch_types=[
        pltpu.VMEM((NCHUNK, CH), jnp.int32),
        pltpu.VMEM((CH, 16), jnp.float32),
        pltpu.VMEM((RPT, 16), jnp.float32),
        pltpu.VMEM_SHARED((N2, 16), jnp.float32),
        pltpu.SemaphoreType.DMA,
        pltpu.SemaphoreType.DMA,
    ],
)

_sc_agg = pl.kernel(
    _sc_agg_body,
    out_type=jax.ShapeDtypeStruct((NC, N2, 128), jnp.float32),
    mesh=_SC_MESH,
    compiler_params=_SC_PARAMS,
    scratch_types=[
        pltpu.VMEM((NCHUNK, CH), jnp.int32),
        pltpu.VMEM((NCHUNK, CH), jnp.int32),
        pltpu.VMEM((NBUF, CH, 128), jnp.float32),
        pltpu.VMEM_SHARED((N2, 128), jnp.float32),
        [pltpu.SemaphoreType.DMA] * NBUF,
        [pltpu.SemaphoreType.DMA] * NBUF,
    ],
)


def _sage_layer(s_ref, dinv, h_ref, wl_ref, bl_ref, wr_ref, g_ref, b_ref):
    sp = s_ref[...]
    agg = (sp[0, :N] + sp[1, :N]) * dinv
    h = h_ref[...]
    a = (jnp.dot(agg, wl_ref[...], preferred_element_type=jnp.float32)
         + bl_ref[...]
         + jnp.dot(h, wr_ref[...], preferred_element_type=jnp.float32))
    mean = jnp.mean(a, axis=0, keepdims=True)
    var = jnp.mean((a - mean) ** 2, axis=0, keepdims=True)
    hn = (a - mean) * lax.rsqrt(var + 1e-5) * g_ref[...] + b_ref[...]
    return jnp.maximum(hn, 0.0)


def _tc_l0_body(s_ref, dg_ref, h_ref, wl_ref, bl_ref, wr_ref, g_ref, b_ref,
                hn_ref, dinv_ref):
    dg = dg_ref[...]
    deg = dg[0, :N, 0:1] + dg[1, :N, 0:1]
    dinv = 1.0 / jnp.maximum(deg, 1.0)
    dinv_ref[...] = dinv
    hn_ref[...] = _sage_layer(s_ref, dinv, h_ref, wl_ref, bl_ref, wr_ref,
                              g_ref, b_ref)


def _tc_mid_body(s_ref, dinv_ref, h_ref, wl_ref, bl_ref, wr_ref, g_ref,
                 b_ref, hn_ref):
    hn_ref[...] = _sage_layer(s_ref, dinv_ref[...], h_ref, wl_ref, bl_ref,
                              wr_ref, g_ref, b_ref)


def _tc_fin_body(s_ref, dinv_ref, h_ref, wl_ref, bl_ref, wr_ref, g_ref,
                 b_ref, wc1_ref, bc1_ref, wc2_ref, bc2_ref, o_ref):
    h = _sage_layer(s_ref, dinv_ref[...], h_ref, wl_ref, bl_ref, wr_ref,
                    g_ref, b_ref)
    o1 = jnp.maximum(
        jnp.dot(h, wc1_ref[...], preferred_element_type=jnp.float32)
        + bc1_ref[...], 0.0)
    o_ref[...] = (jnp.dot(o1, wc2_ref[...], preferred_element_type=jnp.float32)
                  + bc2_ref[...])


_f32 = jnp.float32


def _tc_l0(s_par, deg_par, h, wl, bl, wr, g, b):
    return pl.pallas_call(
        _tc_l0_body,
        out_shape=[jax.ShapeDtypeStruct((N, 128), _f32),
                   jax.ShapeDtypeStruct((N, 1), _f32)],
    )(s_par, deg_par, h, wl, bl, wr, g, b)


def _tc_mid(s_par, dinv, h, wl, bl, wr, g, b):
    return pl.pallas_call(
        _tc_mid_body,
        out_shape=jax.ShapeDtypeStruct((N, 128), _f32),
    )(s_par, dinv, h, wl, bl, wr, g, b)


def _tc_fin(s_par, dinv, h, wl, bl, wr, g, b, wc1, bc1, wc2, bc2):
    return pl.pallas_call(
        _tc_fin_body,
        out_shape=jax.ShapeDtypeStruct((N, 1), _f32),
    )(s_par, dinv, h, wl, bl, wr, g, b, wc1, bc1, wc2, bc2)


def kernel(x, edge_index, Wl0, bl0, Wr0, gamma0, beta0, Wl1, bl1, Wr1,
           gamma1, beta1, Wl2, bl2, Wr2, gamma2, beta2, Wc1, bc1, Wc2, bc2):
    src = edge_index[0].reshape(NW, NCHUNK, CH)
    dst = edge_index[1].reshape(NW, NCHUNK, CH)
    row = lambda v: v.reshape(1, -1)

    deg_par = _sc_deg(dst)
    s_par = _sc_agg(x, src, dst)
    h, dinv = _tc_l0(s_par, deg_par, x, Wl0, row(bl0), Wr0,
                     row(gamma0), row(beta0))
    s_par = _sc_agg(h, src, dst)
    h = _tc_mid(s_par, dinv, h, Wl1, row(bl1), Wr1, row(gamma1), row(beta1))
    s_par = _sc_agg(h, src, dst)
    out = _tc_fin(s_par, dinv, h, Wl2, row(bl2), Wr2, row(gamma2),
                  row(beta2), Wc1, row(bc1), Wc2, bc2.reshape(1, 1))
    return out[:, 0]


# speedup vs baseline: 1.0445x; 1.0093x over previous
import functools

import jax
import jax.numpy as jnp
from jax import lax
from jax.experimental import pallas as pl
from jax.experimental.pallas import tpu as pltpu
from jax.experimental.pallas import tpu_sc as plsc

N = 10000
E = 320000
D = 128
NC = 2
NS = 16
NW = NC * NS
EPW = E // NW
CH = 40
NCHUNK = EPW // CH
NBUF = 5
NROUND = NCHUNK // NBUF
DCH = 80
DNCHUNK = EPW // DCH
N2 = 10240
RPT = N2 // NS


def _sc_agg_body(m_hbm, src_hbm, dst_hbm, s_out,
                 src_v, dst_v, rows_v, acc_sh, semg, sems):
    c = lax.axis_index("c")
    s = lax.axis_index("s")
    wid = c * NS + s

    pltpu.sync_copy(src_hbm.at[wid], src_v)
    pltpu.sync_copy(dst_hbm.at[wid], dst_v)

    def _g_start(ci, b):
        pltpu.async_copy(m_hbm.at[src_v.at[ci]], rows_v.at[b], semg[b])

    def _g_wait(b):
        pltpu.make_async_copy(m_hbm.at[pl.ds(0, CH)], rows_v.at[b],
                              semg[b]).wait()

    def _s_start(ci, b):
        pltpu.async_copy(rows_v.at[b], acc_sh.at[dst_v.at[ci]], sems[b],
                         add=True)

    def _s_wait(b):
        pltpu.make_async_copy(rows_v.at[b], acc_sh.at[pl.ds(0, CH)],
                              sems[b]).wait()

    for b in range(1, NBUF):
        _g_start(b, b)

    def _zrow(i, _):
        for j in range(8):
            rows_v[0, i, pl.ds(j * 16, 16)] = jnp.zeros((16,), jnp.float32)
        return 0
    lax.fori_loop(0, CH, _zrow, 0)
    for k in range(RPT // CH):
        pltpu.sync_copy(rows_v.at[0], acc_sh.at[pl.ds(s * RPT + k * CH, CH)])

    plsc.subcore_barrier()

    _g_start(0, 0)

    def _round(k, _):
        c0 = k * NBUF
        for b in range(NBUF):
            _g_wait(b)
            _s_start(c0 + b, b)
        for b in range(NBUF):
            _s_wait(b)
            _g_start(c0 + NBUF + b, b)
        return 0
    lax.fori_loop(0, NROUND - 1, _round, 0)

    c0 = (NROUND - 1) * NBUF
    for b in range(NBUF):
        _g_wait(b)
        _s_start(c0 + b, b)
    for b in range(NBUF):
        _s_wait(b)

    plsc.subcore_barrier()

    pltpu.sync_copy(acc_sh.at[pl.ds(s * RPT, RPT)],
                    s_out.at[c, pl.ds(s * RPT, RPT)])


def _sc_deg_body(dst_hbm, deg_out, dst_v, ones_v, zbufd_v, dega_sh,
                 semd0, semd1):
    c = lax.axis_index("c")
    s = lax.axis_index("s")
    wid = c * NS + s

    def _zdrow(i, _):
        zbufd_v[i, pl.ds(0, 16)] = jnp.zeros((16,), jnp.float32)
        return 0
    lax.fori_loop(0, RPT, _zdrow, 0)
    pltpu.sync_copy(zbufd_v, dega_sh.at[pl.ds(s * RPT, RPT)])

    def _orow(i, _):
        ones_v[i, pl.ds(0, 16)] = jnp.ones((16,), jnp.float32)
        return 0
    lax.fori_loop(0, DCH, _orow, 0)

    pltpu.sync_copy(dst_hbm.at[wid], dst_v)

    plsc.subcore_barrier()

    def _d_start(ci, sem):
        pltpu.async_copy(ones_v, dega_sh.at[dst_v.at[ci]], sem, add=True)

    def _d_wait(sem):
        pltpu.make_async_copy(ones_v, dega_sh.at[pl.ds(0, DCH)], sem).wait()

    _d_start(0, semd0)

    def _chunk(k, _):
        c0 = k * 2
        _d_start(c0 + 1, semd1)
        _d_wait(semd0)
        _d_start(c0 + 2, semd0)
        _d_wait(semd1)
        return 0
    lax.fori_loop(0, (DNCHUNK - 2) // 2, _chunk, 0)

    _d_start(DNCHUNK - 1, semd1)
    _d_wait(semd0)
    _d_wait(semd1)

    plsc.subcore_barrier()

    pltpu.sync_copy(dega_sh.at[pl.ds(s * RPT, RPT)],
                    deg_out.at[c, pl.ds(s * RPT, RPT)])


_SC_MESH = plsc.VectorSubcoreMesh(core_axis_name="c", subcore_axis_name="s")
_SC_PARAMS = pltpu.CompilerParams(use_tc_tiling_on_sc=False)

_sc_deg = pl.kernel(
    _sc_deg_body,
    out_type=jax.ShapeDtypeStruct((NC, N2, 16), jnp.float32),
    mesh=_SC_MESH,
    compiler_params=_SC_PARAMS,
    scratch_types=[
        pltpu.VMEM((DNCHUNK, DCH), jnp.int32),
        pltpu.VMEM((DCH, 16), jnp.float32),
        pltpu.VMEM((RPT, 16), jnp.float32),
        pltpu.VMEM_SHARED((N2, 16), jnp.float32),
        pltpu.SemaphoreType.DMA,
        pltpu.SemaphoreType.DMA,
    ],
)

_sc_agg = pl.kernel(
    _sc_agg_body,
    out_type=jax.ShapeDtypeStruct((NC, N2, 128), jnp.float32),
    mesh=_SC_MESH,
    compiler_params=_SC_PARAMS,
    scratch_types=[
        pltpu.VMEM((NCHUNK, CH), jnp.int32),
        pltpu.VMEM((NCHUNK, CH), jnp.int32),
        pltpu.VMEM((NBUF, CH, 128), jnp.float32),
        pltpu.VMEM_SHARED((N2, 128), jnp.float32),
        [pltpu.SemaphoreType.DMA] * NBUF,
        [pltpu.SemaphoreType.DMA] * NBUF,
    ],
)


def _sage_layer(s_ref, dinv, h_ref, wl_ref, bl_ref, wr_ref, g_ref, b_ref):
    sp = s_ref[...]
    agg = (sp[0, :N] + sp[1, :N]) * dinv
    h = h_ref[...]
    a = (jnp.dot(agg, wl_ref[...], preferred_element_type=jnp.float32)
         + bl_ref[...]
         + jnp.dot(h, wr_ref[...], preferred_element_type=jnp.float32))
    mean = jnp.mean(a, axis=0, keepdims=True)
    var = jnp.mean((a - mean) ** 2, axis=0, keepdims=True)
    hn = (a - mean) * lax.rsqrt(var + 1e-5) * g_ref[...] + b_ref[...]
    return jnp.maximum(hn, 0.0)


def _tc_l0_body(s_ref, dg_ref, h_ref, wl_ref, bl_ref, wr_ref, g_ref, b_ref,
                hn_ref, dinv_ref):
    dg = dg_ref[...]
    deg = dg[0, :N, 0:1] + dg[1, :N, 0:1]
    dinv = 1.0 / jnp.maximum(deg, 1.0)
    dinv_ref[...] = dinv
    hn_ref[...] = _sage_layer(s_ref, dinv, h_ref, wl_ref, bl_ref, wr_ref,
                              g_ref, b_ref)


def _tc_mid_body(s_ref, dinv_ref, h_ref, wl_ref, bl_ref, wr_ref, g_ref,
                 b_ref, hn_ref):
    hn_ref[...] = _sage_layer(s_ref, dinv_ref[...], h_ref, wl_ref, bl_ref,
                              wr_ref, g_ref, b_ref)


def _tc_fin_body(s_ref, dinv_ref, h_ref, wl_ref, bl_ref, wr_ref, g_ref,
                 b_ref, wc1_ref, bc1_ref, wc2_ref, bc2_ref, o_ref):
    h = _sage_layer(s_ref, dinv_ref[...], h_ref, wl_ref, bl_ref, wr_ref,
                    g_ref, b_ref)
    o1 = jnp.maximum(
        jnp.dot(h, wc1_ref[...], preferred_element_type=jnp.float32)
        + bc1_ref[...], 0.0)
    o_ref[...] = (jnp.dot(o1, wc2_ref[...], preferred_element_type=jnp.float32)
                  + bc2_ref[...])


_f32 = jnp.float32


def _tc_l0(s_par, deg_par, h, wl, bl, wr, g, b):
    return pl.pallas_call(
        _tc_l0_body,
        out_shape=[jax.ShapeDtypeStruct((N, 128), _f32),
                   jax.ShapeDtypeStruct((N, 1), _f32)],
    )(s_par, deg_par, h, wl, bl, wr, g, b)


def _tc_mid(s_par, dinv, h, wl, bl, wr, g, b):
    return pl.pallas_call(
        _tc_mid_body,
        out_shape=jax.ShapeDtypeStruct((N, 128), _f32),
    )(s_par, dinv, h, wl, bl, wr, g, b)


def _tc_fin(s_par, dinv, h, wl, bl, wr, g, b, wc1, bc1, wc2, bc2):
    return pl.pallas_call(
        _tc_fin_body,
        out_shape=jax.ShapeDtypeStruct((N, 1), _f32),
    )(s_par, dinv, h, wl, bl, wr, g, b, wc1, bc1, wc2, bc2)


def kernel(x, edge_index, Wl0, bl0, Wr0, gamma0, beta0, Wl1, bl1, Wr1,
           gamma1, beta1, Wl2, bl2, Wr2, gamma2, beta2, Wc1, bc1, Wc2, bc2):
    src = edge_index[0].reshape(NW, NCHUNK, CH)
    dst = edge_index[1].reshape(NW, NCHUNK, CH)
    dst_w = edge_index[1].reshape(NW, DNCHUNK, DCH)
    row = lambda v: v.reshape(1, -1)

    deg_par = _sc_deg(dst_w)
    s_par = _sc_agg(x, src, dst)
    h, dinv = _tc_l0(s_par, deg_par, x, Wl0, row(bl0), Wr0,
                     row(gamma0), row(beta0))
    s_par = _sc_agg(h, src, dst)
    h = _tc_mid(s_par, dinv, h, Wl1, row(bl1), Wr1, row(gamma1), row(beta1))
    s_par = _sc_agg(h, src, dst)
    out = _tc_fin(s_par, dinv, h, Wl2, row(bl2), Wr2, row(gamma2),
                  row(beta2), Wc1, row(bc1), Wc2, bc2.reshape(1, 1))
    return out[:, 0]
